# baseline (speedup 1.0000x reference)
import functools

import jax
import jax.numpy as jnp
from jax import lax
from jax.experimental import pallas as pl
from jax.experimental.pallas import tpu as pltpu
from jax.experimental.pallas import tpu_sc as plsc

_N = 4096
_D = 64
_KTOP = 167772
_TOTAL = _N * _N

_NC = 2
_NS = 16
_NW = _NC * _NS
_ROWS_W = _N // _NW
_WROWS = 8
_NWIN = _ROWS_W // _WROWS
_UNROLL = 16


def _v_body(m1_ref, m2_ref, v_ref):
    s = lax.dot_general(m1_ref[...], m2_ref[...], (((1,), (1,)), ((), ())),
                        preferred_element_type=jnp.float32)
    v_ref[...] = jnp.maximum(s, 0.0)


def _compute_v(M1, M2):
    blk = 256
    return pl.pallas_call(
        _v_body,
        grid=(_N // blk,),
        in_specs=[pl.BlockSpec((blk, _D), lambda i: (i, 0)),
                  pl.BlockSpec((_N, _D), lambda i: (0, 0))],
        out_specs=pl.BlockSpec((blk, _N), lambda i: (i, 0)),
        out_shape=jax.ShapeDtypeStruct((_N, _N), jnp.float32),
    )(M1, M2)


def _make_hist(nbins, mode):
    hist_words = nbins * 16
    hcols = hist_words // 8
    colbits = hcols.bit_length() - 1
    mesh = plsc.VectorSubcoreMesh(core_axis_name="c", subcore_axis_name="s")
    with_sel = mode != 1

    def body(*refs):
        if with_sel:
            v_hbm, r_hbm, out_hbm, win0, win1, hist_v, sel_v, sem0, sem1 = refs
        else:
            v_hbm, out_hbm, win0, win1, hist_v, sem0, sem1 = refs
        wid = lax.axis_index("s") * _NC + lax.axis_index("c")
        row_base = wid * _ROWS_W
        lane = lax.iota(jnp.int32, 16)
        ones = jnp.ones((16,), jnp.int32)

        @plsc.parallel_loop(0, hist_words // 16, unroll=_UNROLL)
        def _(i):
            hist_v[pl.ds(i * 16, 16)] = jnp.zeros((16,), jnp.int32)

        if with_sel:
            pltpu.sync_copy(r_hbm.at[0, pl.ds(0, 16)], sel_v)
            sel = sel_v[...]
            sel20 = lax.shift_left(sel, 20)
        else:
            sel = sel20 = None

        def process(win):
            for r in range(_WROWS):
                @plsc.parallel_loop(0, _N // 16, unroll=_UNROLL)
                def _(i, r=r):
                    v = win[r, pl.ds(i * 16, 16)]
                    bits = lax.bitcast_convert_type(v, jnp.int32)
                    if mode == 1:
                        bucket = lax.shift_right_logical(bits, 20)
                        mk = None
                    elif mode == 2:
                        bucket = jnp.bitwise_and(
                            lax.shift_right_logical(bits, 9), 0x7FF)
                        mk = lax.shift_right_logical(bits, 20) == sel
                    else:
                        bucket = jnp.bitwise_and(bits, 0x1FF)
                        mk = lax.shift_right_logical(bits, 9) == sel
                    idx = jnp.bitwise_or(lax.shift_left(bucket, 4), lane)
                    plsc.addupdate_scatter(hist_v, [idx], ones, mask=mk)

        def dma(w, buf, sem):
            return pltpu.make_async_copy(
                v_hbm.at[pl.ds(row_base + w * _WROWS, _WROWS), :], buf, sem)

        dma(0, win0, sem0).start()

        def pair_body(p, _):
            w0 = 2 * p
            dma(w0 + 1, win1, sem1).start()
            dma(w0, win0, sem0).wait()
            process(win0)

            @pl.when(p < _NWIN // 2 - 1)
            def _():
                dma(w0 + 2, win0, sem0).start()

            dma(w0 + 1, win1, sem1).wait()
            process(win1)
            return 0
        lax.fori_loop(0, _NWIN // 2, pair_body, 0)

        for r in range(8):
            pltpu.sync_copy(hist_v.at[pl.ds(r * hcols, hcols)],
                            out_hbm.at[wid * 8 + r, :])

    scratch = [pltpu.VMEM((_WROWS, _N), jnp.float32),
               pltpu.VMEM((_WROWS, _N), jnp.float32),
               pltpu.VMEM((hist_words,), jnp.int32)]
    if with_sel:
        scratch.append(pltpu.VMEM((16,), jnp.int32))
    scratch += [pltpu.SemaphoreType.DMA, pltpu.SemaphoreType.DMA]
    return pl.kernel(
        body,
        out_type=jax.ShapeDtypeStruct((_NW * 8, hcols), jnp.int32),
        mesh=mesh,
        scratch_types=scratch,
        compiler_params=pltpu.CompilerParams(
            needs_layout_passes=False, use_tc_tiling_on_sc=True),
    )


def _search(h, jb, nbits, k):
    p = jnp.int32(0)
    for i in range(nbits):
        c = p + jnp.int32(1 << (nbits - 1 - i))
        ic = jnp.sum(jnp.where(jb >= c, h, 0))
        p = jnp.where(ic >= k, c, p)
    ca = jnp.sum(jnp.where(jb > p, h, 0))
    return p, k - ca


def _hist_slab(h_ref):
    full = h_ref[...]
    rows, hcols = full.shape
    h = jnp.sum(jnp.reshape(full, (rows // 8, 8, hcols)), axis=0)
    r = lax.broadcasted_iota(jnp.int32, (8, hcols), 0)
    c = lax.broadcasted_iota(jnp.int32, (8, hcols), 1)
    jb = lax.shift_right_logical(r * hcols + c, 4)
    return h, jb


def _r1_body(h_ref, out_ref):
    h, jb = _hist_slab(h_ref)
    b, kp = _search(h, jb, 11, jnp.int32(_KTOP))
    row = lax.broadcasted_iota(jnp.int32, (8, 128), 0)
    out_ref[...] = jnp.where(row == 0, b, kp)


def _r2_body(h_ref, r_ref, out_ref):
    b1 = r_ref[0, 0]
    k1 = r_ref[1, 0]
    h, jb = _hist_slab(h_ref)
    b2, k2 = _search(h, jb, 11, k1)
    c2 = jnp.bitwise_or(lax.shift_left(b1, 11), b2)
    row = lax.broadcasted_iota(jnp.int32, (8, 128), 0)
    out_ref[...] = jnp.where(row == 0, c2, k2)


def _reduce1(h1):
    return pl.pallas_call(
        _r1_body,
        out_shape=jax.ShapeDtypeStruct((8, 128), jnp.int32),
    )(h1)


def _reduce2(h2, r1):
    return pl.pallas_call(
        _r2_body,
        out_shape=jax.ShapeDtypeStruct((8, 128), jnp.int32),
    )(h2, r1)


def _mask_body(m1_ref, m2_ref, h_ref, r_ref, out_ref, thr_ref):
    @pl.when(pl.program_id(0) == 0)
    def _():
        c2 = r_ref[0, 0]
        k2 = r_ref[1, 0]
        h, jb = _hist_slab(h_ref)
        b3, _ = _search(h, jb, 9, k2)
        thr_ref[0] = jnp.bitwise_or(lax.shift_left(c2, 9), b3)

    s = lax.dot_general(m1_ref[...], m2_ref[...], (((1,), (1,)), ((), ())),
                        preferred_element_type=jnp.float32)
    a = jax.nn.sigmoid(jnp.maximum(s, 0.0))
    kth = jax.nn.sigmoid(lax.bitcast_convert_type(
        jnp.full((1, 1), thr_ref[0], jnp.int32), jnp.float32))
    out_ref[...] = jnp.where(a > kth, a, 0.0)


def _apply_mask(M1, M2, h3, r2):
    blk = 256
    return pl.pallas_call(
        _mask_body,
        grid=(_N // blk,),
        in_specs=[pl.BlockSpec((blk, _D), lambda i: (i, 0)),
                  pl.BlockSpec((_N, _D), lambda i: (0, 0)),
                  pl.BlockSpec((_NW * 8, 1024), lambda i: (0, 0)),
                  pl.BlockSpec((8, 128), lambda i: (0, 0))],
        out_specs=pl.BlockSpec((blk, _N), lambda i: (i, 0)),
        out_shape=jax.ShapeDtypeStruct((_N, _N), jnp.float32),
        scratch_shapes=[pltpu.SMEM((1,), jnp.int32)],
    )(M1, M2, h3, r2)


_hist1 = _make_hist(2048, 1)
_hist2 = _make_hist(2048, 2)
_hist3 = _make_hist(512, 3)


def kernel(x, M1, M2):
    del x
    V = _compute_v(M1, M2)
    h1 = _hist1(V)
    r1 = _reduce1(h1)
    h2 = _hist2(V, r1)
    r2 = _reduce2(h2, r1)
    h3 = _hist3(V, r2)
    return _apply_mask(M1, M2, h3, r2)


# speedup vs baseline: 1.0172x; 1.0172x over previous
import functools

import jax
import jax.numpy as jnp
from jax import lax
from jax.experimental import pallas as pl
from jax.experimental.pallas import tpu as pltpu
from jax.experimental.pallas import tpu_sc as plsc

_N = 4096
_D = 64
_KTOP = 167772
_TOTAL = _N * _N

_NC = 2
_NS = 16
_NW = _NC * _NS
_ROWS_W = _N // _NW
_WROWS = 8
_NWIN = _ROWS_W // _WROWS
_UNROLL = 8


def _v_body(m1_ref, m2_ref, v_ref):
    s = lax.dot_general(m1_ref[...], m2_ref[...], (((1,), (1,)), ((), ())),
                        preferred_element_type=jnp.float32)
    v_ref[...] = jnp.maximum(s, 0.0)


def _compute_v(M1, M2):
    blk = 256
    return pl.pallas_call(
        _v_body,
        grid=(_N // blk,),
        in_specs=[pl.BlockSpec((blk, _D), lambda i: (i, 0)),
                  pl.BlockSpec((_N, _D), lambda i: (0, 0))],
        out_specs=pl.BlockSpec((blk, _N), lambda i: (i, 0)),
        out_shape=jax.ShapeDtypeStruct((_N, _N), jnp.float32),
    )(M1, M2)


def _make_hist(nbins, mode):
    hist_words = nbins * 16
    hcols = hist_words // 8
    colbits = hcols.bit_length() - 1
    mesh = plsc.VectorSubcoreMesh(core_axis_name="c", subcore_axis_name="s")
    with_sel = mode != 1

    def body(*refs):
        if with_sel:
            v_hbm, r_hbm, out_hbm, win0, win1, hist_v, sel_v, sem0, sem1 = refs
        else:
            v_hbm, out_hbm, win0, win1, hist_v, sem0, sem1 = refs
        wid = lax.axis_index("s") * _NC + lax.axis_index("c")
        row_base = wid * _ROWS_W
        lane = lax.iota(jnp.int32, 16)
        ones = jnp.ones((16,), jnp.int32)

        @plsc.parallel_loop(0, hist_words // 16, unroll=_UNROLL)
        def _(i):
            hist_v[pl.ds(i * 16, 16)] = jnp.zeros((16,), jnp.int32)

        if with_sel:
            pltpu.sync_copy(r_hbm.at[0, pl.ds(0, 16)], sel_v)
            sel = sel_v[...]
            sel20 = lax.shift_left(sel, 20)
        else:
            sel = sel20 = None

        def process(win):
            @plsc.parallel_loop(0, _N // 16, unroll=2)
            def _(i):
                for r in range(_WROWS):
                    v = win[r, pl.ds(i * 16, 16)]
                    bits = lax.bitcast_convert_type(v, jnp.int32)
                    if mode == 1:
                        bucket = lax.shift_right_logical(bits, 20)
                        mk = None
                    elif mode == 2:
                        bucket = jnp.bitwise_and(
                            lax.shift_right_logical(bits, 9), 0x7FF)
                        mk = lax.shift_right_logical(bits, 20) == sel
                    else:
                        bucket = jnp.bitwise_and(bits, 0x1FF)
                        mk = lax.shift_right_logical(bits, 9) == sel
                    idx = jnp.bitwise_or(lax.shift_left(bucket, 4), lane)
                    plsc.addupdate_scatter(hist_v, [idx], ones, mask=mk)

        def dma(w, buf, sem):
            return pltpu.make_async_copy(
                v_hbm.at[pl.ds(row_base + w * _WROWS, _WROWS), :], buf, sem)

        dma(0, win0, sem0).start()

        def pair_body(p, _):
            w0 = 2 * p
            dma(w0 + 1, win1, sem1).start()
            dma(w0, win0, sem0).wait()
            process(win0)

            @pl.when(p < _NWIN // 2 - 1)
            def _():
                dma(w0 + 2, win0, sem0).start()

            dma(w0 + 1, win1, sem1).wait()
            process(win1)
            return 0
        lax.fori_loop(0, _NWIN // 2, pair_body, 0)

        for r in range(8):
            pltpu.sync_copy(hist_v.at[pl.ds(r * hcols, hcols)],
                            out_hbm.at[wid * 8 + r, :])

    scratch = [pltpu.VMEM((_WROWS, _N), jnp.float32),
               pltpu.VMEM((_WROWS, _N), jnp.float32),
               pltpu.VMEM((hist_words,), jnp.int32)]
    if with_sel:
        scratch.append(pltpu.VMEM((16,), jnp.int32))
    scratch += [pltpu.SemaphoreType.DMA, pltpu.SemaphoreType.DMA]
    return pl.kernel(
        body,
        out_type=jax.ShapeDtypeStruct((_NW * 8, hcols), jnp.int32),
        mesh=mesh,
        scratch_types=scratch,
        compiler_params=pltpu.CompilerParams(
            needs_layout_passes=False, use_tc_tiling_on_sc=True),
    )


def _search(h, jb, nbits, k):
    p = jnp.int32(0)
    for i in range(nbits):
        c = p + jnp.int32(1 << (nbits - 1 - i))
        ic = jnp.sum(jnp.where(jb >= c, h, 0))
        p = jnp.where(ic >= k, c, p)
    ca = jnp.sum(jnp.where(jb > p, h, 0))
    return p, k - ca


def _hist_slab(h_ref):
    full = h_ref[...]
    rows, hcols = full.shape
    h = jnp.sum(jnp.reshape(full, (rows // 8, 8, hcols)), axis=0)
    r = lax.broadcasted_iota(jnp.int32, (8, hcols), 0)
    c = lax.broadcasted_iota(jnp.int32, (8, hcols), 1)
    jb = lax.shift_right_logical(r * hcols + c, 4)
    return h, jb


def _r1_body(h_ref, out_ref):
    h, jb = _hist_slab(h_ref)
    b, kp = _search(h, jb, 11, jnp.int32(_KTOP))
    row = lax.broadcasted_iota(jnp.int32, (8, 128), 0)
    out_ref[...] = jnp.where(row == 0, b, kp)


def _r2_body(h_ref, r_ref, out_ref):
    b1 = r_ref[0, 0]
    k1 = r_ref[1, 0]
    h, jb = _hist_slab(h_ref)
    b2, k2 = _search(h, jb, 11, k1)
    c2 = jnp.bitwise_or(lax.shift_left(b1, 11), b2)
    row = lax.broadcasted_iota(jnp.int32, (8, 128), 0)
    out_ref[...] = jnp.where(row == 0, c2, k2)


def _reduce1(h1):
    return pl.pallas_call(
        _r1_body,
        out_shape=jax.ShapeDtypeStruct((8, 128), jnp.int32),
    )(h1)


def _reduce2(h2, r1):
    return pl.pallas_call(
        _r2_body,
        out_shape=jax.ShapeDtypeStruct((8, 128), jnp.int32),
    )(h2, r1)


def _mask_body(m1_ref, m2_ref, h_ref, r_ref, out_ref, thr_ref):
    @pl.when(pl.program_id(0) == 0)
    def _():
        c2 = r_ref[0, 0]
        k2 = r_ref[1, 0]
        h, jb = _hist_slab(h_ref)
        b3, _ = _search(h, jb, 9, k2)
        thr_ref[0] = jnp.bitwise_or(lax.shift_left(c2, 9), b3)

    s = lax.dot_general(m1_ref[...], m2_ref[...], (((1,), (1,)), ((), ())),
                        preferred_element_type=jnp.float32)
    a = jax.nn.sigmoid(jnp.maximum(s, 0.0))
    kth = jax.nn.sigmoid(lax.bitcast_convert_type(
        jnp.full((1, 1), thr_ref[0], jnp.int32), jnp.float32))
    out_ref[...] = jnp.where(a > kth, a, 0.0)


def _apply_mask(M1, M2, h3, r2):
    blk = 256
    return pl.pallas_call(
        _mask_body,
        grid=(_N // blk,),
        in_specs=[pl.BlockSpec((blk, _D), lambda i: (i, 0)),
                  pl.BlockSpec((_N, _D), lambda i: (0, 0)),
                  pl.BlockSpec((_NW * 8, 1024), lambda i: (0, 0)),
                  pl.BlockSpec((8, 128), lambda i: (0, 0))],
        out_specs=pl.BlockSpec((blk, _N), lambda i: (i, 0)),
        out_shape=jax.ShapeDtypeStruct((_N, _N), jnp.float32),
        scratch_shapes=[pltpu.SMEM((1,), jnp.int32)],
    )(M1, M2, h3, r2)


_hist1 = _make_hist(2048, 1)
_hist2 = _make_hist(2048, 2)
_hist3 = _make_hist(512, 3)


def kernel(x, M1, M2):
    del x
    V = _compute_v(M1, M2)
    h1 = _hist1(V)
    r1 = _reduce1(h1)
    h2 = _hist2(V, r1)
    r2 = _reduce2(h2, r1)
    h3 = _hist3(V, r2)
    return _apply_mask(M1, M2, h3, r2)


# speedup vs baseline: 1.0416x; 1.0239x over previous
import functools

import jax
import jax.numpy as jnp
from jax import lax
from jax.experimental import pallas as pl
from jax.experimental.pallas import tpu as pltpu
from jax.experimental.pallas import tpu_sc as plsc

_N = 4096
_D = 64
_KTOP = 167772
_TOTAL = _N * _N

_NC = 2
_NS = 16
_NW = _NC * _NS
_ROWS_W = _N // _NW
_WROWS = 8
_NWIN = _ROWS_W // _WROWS
_UNROLL = 8


def _v_body(m1_ref, m2_ref, v_ref):
    s = lax.dot_general(m1_ref[...], m2_ref[...], (((1,), (1,)), ((), ())),
                        preferred_element_type=jnp.float32)
    v_ref[...] = jnp.maximum(s, 0.0)


def _compute_v(M1, M2):
    blk = 512
    return pl.pallas_call(
        _v_body,
        grid=(_N // blk,),
        in_specs=[pl.BlockSpec((blk, _D), lambda i: (i, 0)),
                  pl.BlockSpec((_N, _D), lambda i: (0, 0))],
        out_specs=pl.BlockSpec((blk, _N), lambda i: (i, 0)),
        out_shape=jax.ShapeDtypeStruct((_N, _N), jnp.float32),
    )(M1, M2)


def _make_hist(nbins, mode):
    hist_words = nbins * 16
    hcols = hist_words // 8
    colbits = hcols.bit_length() - 1
    mesh = plsc.VectorSubcoreMesh(core_axis_name="c", subcore_axis_name="s")
    with_sel = mode != 1

    def body(*refs):
        if with_sel:
            v_hbm, r_hbm, out_hbm, win0, win1, hist_v, sel_v, sem0, sem1 = refs
        else:
            v_hbm, out_hbm, win0, win1, hist_v, sem0, sem1 = refs
        wid = lax.axis_index("s") * _NC + lax.axis_index("c")
        row_base = wid * _ROWS_W
        lane = lax.iota(jnp.int32, 16)
        ones = jnp.ones((16,), jnp.int32)

        @plsc.parallel_loop(0, hist_words // 16, unroll=_UNROLL)
        def _(i):
            hist_v[pl.ds(i * 16, 16)] = jnp.zeros((16,), jnp.int32)

        if with_sel:
            pltpu.sync_copy(r_hbm.at[0, pl.ds(0, 16)], sel_v)
            sel = sel_v[...]
            sel20 = lax.shift_left(sel, 20)
        else:
            sel = sel20 = None

        def process(win):
            @plsc.parallel_loop(0, _N // 16, unroll=2)
            def _(i):
                for r in range(_WROWS):
                    v = win[r, pl.ds(i * 16, 16)]
                    bits = lax.bitcast_convert_type(v, jnp.int32)
                    if mode == 1:
                        bucket = lax.shift_right_logical(bits, 20)
                        mk = None
                    elif mode == 2:
                        bucket = jnp.bitwise_and(
                            lax.shift_right_logical(bits, 9), 0x7FF)
                        mk = lax.shift_right_logical(bits, 20) == sel
                    else:
                        bucket = jnp.bitwise_and(bits, 0x1FF)
                        mk = lax.shift_right_logical(bits, 9) == sel
                    idx = jnp.bitwise_or(lax.shift_left(bucket, 4), lane)
                    plsc.addupdate_scatter(hist_v, [idx], ones, mask=mk)

        def dma(w, buf, sem):
            return pltpu.make_async_copy(
                v_hbm.at[pl.ds(row_base + w * _WROWS, _WROWS), :], buf, sem)

        dma(0, win0, sem0).start()

        def pair_body(p, _):
            w0 = 2 * p
            dma(w0 + 1, win1, sem1).start()
            dma(w0, win0, sem0).wait()
            process(win0)

            @pl.when(p < _NWIN // 2 - 1)
            def _():
                dma(w0 + 2, win0, sem0).start()

            dma(w0 + 1, win1, sem1).wait()
            process(win1)
            return 0
        lax.fori_loop(0, _NWIN // 2, pair_body, 0)

        for r in range(8):
            pltpu.sync_copy(hist_v.at[pl.ds(r * hcols, hcols)],
                            out_hbm.at[wid * 8 + r, :])

    scratch = [pltpu.VMEM((_WROWS, _N), jnp.float32),
               pltpu.VMEM((_WROWS, _N), jnp.float32),
               pltpu.VMEM((hist_words,), jnp.int32)]
    if with_sel:
        scratch.append(pltpu.VMEM((16,), jnp.int32))
    scratch += [pltpu.SemaphoreType.DMA, pltpu.SemaphoreType.DMA]
    return pl.kernel(
        body,
        out_type=jax.ShapeDtypeStruct((_NW * 8, hcols), jnp.int32),
        mesh=mesh,
        scratch_types=scratch,
        compiler_params=pltpu.CompilerParams(
            needs_layout_passes=False, use_tc_tiling_on_sc=True),
    )


def _search(h, jb, nbits, k):
    p = jnp.int32(0)
    for i in range(nbits):
        c = p + jnp.int32(1 << (nbits - 1 - i))
        ic = jnp.sum(jnp.where(jb >= c, h, 0))
        p = jnp.where(ic >= k, c, p)
    ca = jnp.sum(jnp.where(jb > p, h, 0))
    return p, k - ca


def _hist_slab(h_ref):
    full = h_ref[...]
    rows, hcols = full.shape
    h = jnp.sum(jnp.reshape(full, (rows // 8, 8, hcols)), axis=0)
    r = lax.broadcasted_iota(jnp.int32, (8, hcols), 0)
    c = lax.broadcasted_iota(jnp.int32, (8, hcols), 1)
    jb = lax.shift_right_logical(r * hcols + c, 4)
    return h, jb


def _r1_body(h_ref, out_ref):
    h, jb = _hist_slab(h_ref)
    b, kp = _search(h, jb, 11, jnp.int32(_KTOP))
    row = lax.broadcasted_iota(jnp.int32, (8, 128), 0)
    out_ref[...] = jnp.where(row == 0, b, kp)


def _r2_body(h_ref, r_ref, out_ref):
    b1 = r_ref[0, 0]
    k1 = r_ref[1, 0]
    h, jb = _hist_slab(h_ref)
    b2, k2 = _search(h, jb, 11, k1)
    c2 = jnp.bitwise_or(lax.shift_left(b1, 11), b2)
    row = lax.broadcasted_iota(jnp.int32, (8, 128), 0)
    out_ref[...] = jnp.where(row == 0, c2, k2)


def _reduce1(h1):
    return pl.pallas_call(
        _r1_body,
        out_shape=jax.ShapeDtypeStruct((8, 128), jnp.int32),
    )(h1)


def _reduce2(h2, r1):
    return pl.pallas_call(
        _r2_body,
        out_shape=jax.ShapeDtypeStruct((8, 128), jnp.int32),
    )(h2, r1)


def _mask_body(m1_ref, m2_ref, h_ref, r_ref, out_ref, thr_ref):
    @pl.when(pl.program_id(0) == 0)
    def _():
        c2 = r_ref[0, 0]
        k2 = r_ref[1, 0]
        h, jb = _hist_slab(h_ref)
        b3, _ = _search(h, jb, 9, k2)
        thr_ref[0] = jnp.bitwise_or(lax.shift_left(c2, 9), b3)

    s = lax.dot_general(m1_ref[...], m2_ref[...], (((1,), (1,)), ((), ())),
                        preferred_element_type=jnp.float32)
    a = jax.nn.sigmoid(jnp.maximum(s, 0.0))
    kth = jax.nn.sigmoid(lax.bitcast_convert_type(
        jnp.full((1, 1), thr_ref[0], jnp.int32), jnp.float32))
    out_ref[...] = jnp.where(a > kth, a, 0.0)


def _apply_mask(M1, M2, h3, r2):
    blk = 512
    return pl.pallas_call(
        _mask_body,
        grid=(_N // blk,),
        in_specs=[pl.BlockSpec((blk, _D), lambda i: (i, 0)),
                  pl.BlockSpec((_N, _D), lambda i: (0, 0)),
                  pl.BlockSpec((_NW * 8, 1024), lambda i: (0, 0)),
                  pl.BlockSpec((8, 128), lambda i: (0, 0))],
        out_specs=pl.BlockSpec((blk, _N), lambda i: (i, 0)),
        out_shape=jax.ShapeDtypeStruct((_N, _N), jnp.float32),
        scratch_shapes=[pltpu.SMEM((1,), jnp.int32)],
    )(M1, M2, h3, r2)


_hist1 = _make_hist(2048, 1)
_hist2 = _make_hist(2048, 2)
_hist3 = _make_hist(512, 3)


def kernel(x, M1, M2):
    del x
    V = _compute_v(M1, M2)
    h1 = _hist1(V)
    r1 = _reduce1(h1)
    h2 = _hist2(V, r1)
    r2 = _reduce2(h2, r1)
    h3 = _hist3(V, r2)
    return _apply_mask(M1, M2, h3, r2)
